# baseline (speedup 1.0000x reference)
import functools
import math

import jax
import jax.numpy as jnp
from jax import lax
from jax.experimental import pallas as pl
from jax.experimental.pallas import tpu as pltpu
from jax.experimental.pallas import tpu_sc as plsc

_D = 128
_L = 2048
_B = 4
_NC = 2
_NS = 16
_NW = _NC * _NS
_PPW = _L // _NW
_SUB = 2
_RS = _PPW // _SUB
_NCH = _B * _SUB
_LANES = 16
_SCALE = math.sqrt(float(_D))

_mesh = plsc.VectorSubcoreMesh(core_axis_name="c", subcore_axis_name="s")


@jax.jit
def _embed_sc(src, table, pe2):
    @functools.partial(
        pl.kernel,
        out_type=jax.ShapeDtypeStruct((_B, _L, _D), jnp.float32),
        mesh=_mesh,
        scratch_types=[
            pltpu.VMEM((_B, _PPW), jnp.int32),
            [pltpu.VMEM((_RS, _D), jnp.float32) for _ in range(_NCH)],
            [pltpu.VMEM((_RS, _D), jnp.float32) for _ in range(_NCH)],
            [pltpu.SemaphoreType.DMA for _ in range(_NCH)],
            pltpu.SemaphoreType.DMA,
            pltpu.SemaphoreType.DMA,
            pltpu.SemaphoreType.DMA,
        ],
    )
    def k(table_hbm, src_hbm, pe_hbm, out_hbm,
          idx_v, gbufs, obufs, gsems, csem, osem, isem):
        wid = lax.axis_index("s") * _NC + lax.axis_index("c")
        p0 = wid * _PPW

        pe_copies = [
            pltpu.async_copy(
                pe_hbm.at[pl.ds(p0 + h * _RS, _RS)], obufs[b * _SUB + h], csem
            )
            for b in range(_B)
            for h in range(_SUB)
        ]
        idx_copies = [
            pltpu.async_copy(src_hbm.at[b, pl.ds(p0, _PPW)], idx_v.at[b], isem)
            for b in range(_B)
        ]
        for c in idx_copies:
            c.wait()
        gathers = [
            pltpu.async_copy(
                table_hbm.at[idx_v.at[b, pl.ds(h * _RS, _RS)]],
                gbufs[b * _SUB + h],
                gsems[b * _SUB + h],
            )
            for b in range(_B)
            for h in range(_SUB)
        ]
        for c in pe_copies:
            c.wait()

        stores = []
        for b in range(_B):
            for h in range(_SUB):
                ch = b * _SUB + h
                gathers[ch].wait()
                gb, ob = gbufs[ch], obufs[ch]

                @pl.loop(0, _RS)
                def _(r, gb=gb, ob=ob):
                    @pl.loop(0, _D, step=_LANES)
                    def _(c, r=r, gb=gb, ob=ob):
                        sl = (r, pl.ds(c, _LANES))
                        plsc.addupdate(ob.at[sl], gb[sl] * _SCALE)

                stores.append(
                    pltpu.async_copy(
                        ob, out_hbm.at[b, pl.ds(p0 + h * _RS, _RS)], osem
                    )
                )
        for st in stores:
            st.wait()

    return k(table, src, pe2)


def kernel(src, table, pe):
    pe2 = pe.reshape(pe.shape[1], _D)[:_L]
    return _embed_sc(src, table, pe2)


# speedup vs baseline: 1.0452x; 1.0452x over previous
import functools
import math

import jax
import jax.numpy as jnp
from jax import lax
from jax.experimental import pallas as pl
from jax.experimental.pallas import tpu as pltpu
from jax.experimental.pallas import tpu_sc as plsc

_D = 128
_L = 2048
_B = 4
_NC = 2
_NS = 16
_NW = _NC * _NS
_PPW = _L // _NW
_LANES = 16
_SCALE = math.sqrt(float(_D))

_mesh = plsc.VectorSubcoreMesh(core_axis_name="c", subcore_axis_name="s")


@jax.jit
def _embed_sc(src, table, pe2):
    @functools.partial(
        pl.kernel,
        out_type=jax.ShapeDtypeStruct((_B, _L, _D), jnp.float32),
        mesh=_mesh,
        scratch_types=[
            pltpu.VMEM((_B, _PPW), jnp.int32),
            [pltpu.VMEM((_PPW, _D), jnp.float32) for _ in range(_B)],
            pltpu.VMEM((_PPW, _D), jnp.float32),
            [pltpu.SemaphoreType.DMA for _ in range(_B)],
            pltpu.SemaphoreType.DMA,
            pltpu.SemaphoreType.DMA,
            pltpu.SemaphoreType.DMA,
        ],
    )
    def k(table_hbm, src_hbm, pe_hbm, out_hbm,
          idx_v, gbufs, pe_v, gsems, csem, osem, isem):
        wid = lax.axis_index("s") * _NC + lax.axis_index("c")
        p0 = wid * _PPW

        pe_copy = pltpu.async_copy(pe_hbm.at[pl.ds(p0, _PPW)], pe_v, csem)
        idx_copies = [
            pltpu.async_copy(src_hbm.at[b, pl.ds(p0, _PPW)], idx_v.at[b], isem)
            for b in range(_B)
        ]
        for c in idx_copies:
            c.wait()
        gathers = [
            pltpu.async_copy(table_hbm.at[idx_v.at[b]], gbufs[b], gsems[b])
            for b in range(_B)
        ]
        pe_copy.wait()

        stores = []
        for b in range(_B):
            gathers[b].wait()
            gb = gbufs[b]

            @pl.loop(0, _PPW)
            def _(r, gb=gb):
                @pl.loop(0, _D, step=_LANES)
                def _(c, r=r, gb=gb):
                    sl = (r, pl.ds(c, _LANES))
                    gb[sl] = gb[sl] * _SCALE + pe_v[sl]

            stores.append(
                pltpu.async_copy(gb, out_hbm.at[b, pl.ds(p0, _PPW)], osem)
            )
        for st in stores:
            st.wait()

    return k(table, src, pe2)


def kernel(src, table, pe):
    pe2 = pe.reshape(pe.shape[1], _D)[:_L]
    return _embed_sc(src, table, pe2)


# speedup vs baseline: 1.0477x; 1.0025x over previous
import functools
import math

import jax
import jax.numpy as jnp
from jax import lax
from jax.experimental import pallas as pl
from jax.experimental.pallas import tpu as pltpu
from jax.experimental.pallas import tpu_sc as plsc

_D = 128
_L = 2048
_B = 4
_NC = 2
_NS = 16
_NW = _NC * _NS
_PPW = _L // _NW
_LANES = 16
_SCALE = math.sqrt(float(_D))

_mesh = plsc.VectorSubcoreMesh(core_axis_name="c", subcore_axis_name="s")


@jax.jit
def _embed_sc(src, table, pe2):
    @functools.partial(
        pl.kernel,
        out_type=jax.ShapeDtypeStruct((_B, _L, _D), jnp.float32),
        mesh=_mesh,
        scratch_types=[
            pltpu.VMEM((_B, _PPW), jnp.int32),
            [pltpu.VMEM((_PPW, _D), jnp.float32) for _ in range(_B)],
            pltpu.VMEM((_PPW, _D), jnp.float32),
            [pltpu.SemaphoreType.DMA for _ in range(_B)],
            pltpu.SemaphoreType.DMA,
            pltpu.SemaphoreType.DMA,
            pltpu.SemaphoreType.DMA,
        ],
    )
    def k(table_hbm, src_hbm, pe_hbm, out_hbm,
          idx_v, gbufs, pe_v, gsems, csem, osem, isem):
        wid = lax.axis_index("s") * _NC + lax.axis_index("c")
        p0 = wid * _PPW

        idx_copies = [
            pltpu.async_copy(src_hbm.at[b, pl.ds(p0, _PPW)], idx_v.at[b], isem)
            for b in range(_B)
        ]
        pe_copy = pltpu.async_copy(pe_hbm.at[pl.ds(p0, _PPW)], pe_v, csem)
        for c in idx_copies:
            c.wait()
        gathers = [
            pltpu.async_copy(table_hbm.at[idx_v.at[b]], gbufs[b], gsems[b])
            for b in range(_B)
        ]
        pe_copy.wait()

        stores = []
        for b in range(_B):
            gathers[b].wait()
            gb = gbufs[b]

            @pl.loop(0, _PPW)
            def _(r, gb=gb):
                @pl.loop(0, _D, step=_LANES)
                def _(c, r=r, gb=gb):
                    sl = (r, pl.ds(c, _LANES))
                    gb[sl] = gb[sl] * _SCALE + pe_v[sl]

            stores.append(
                pltpu.async_copy(gb, out_hbm.at[b, pl.ds(p0, _PPW)], osem)
            )
        for st in stores:
            st.wait()

    return k(table, src, pe2)


def kernel(src, table, pe):
    pe2 = pe.reshape(pe.shape[1], _D)[:_L]
    return _embed_sc(src, table, pe2)


# speedup vs baseline: 1.0699x; 1.0212x over previous
import functools
import math

import jax
import jax.numpy as jnp
from jax import lax
from jax.experimental import pallas as pl
from jax.experimental.pallas import tpu as pltpu
from jax.experimental.pallas import tpu_sc as plsc

_D = 128
_L = 2048
_B = 4
_NC = 2
_NS = 16
_NW = _NC * _NS
_PPW = _L // _NW
_LANES = 16
_SCALE = math.sqrt(float(_D))

_mesh = plsc.VectorSubcoreMesh(core_axis_name="c", subcore_axis_name="s")


@jax.jit
def _embed_sc(src, table, pe2):
    @functools.partial(
        pl.kernel,
        out_type=jax.ShapeDtypeStruct((_B, _L, _D), jnp.float32),
        mesh=_mesh,
        scratch_types=[
            pltpu.VMEM((_B // 2, 2 * _PPW), jnp.int32),
            [pltpu.VMEM((2 * _PPW, _D), jnp.float32) for _ in range(_B // 2)],
            pltpu.VMEM((_PPW, _D), jnp.float32),
            [pltpu.SemaphoreType.DMA for _ in range(_B // 2)],
            pltpu.SemaphoreType.DMA,
            pltpu.SemaphoreType.DMA,
            pltpu.SemaphoreType.DMA,
        ],
    )
    def k(table_hbm, src_hbm, pe_hbm, out_hbm,
          idx_v, gbufs, pe_v, gsems, csem, osem, isem):
        wid = lax.axis_index("s") * _NC + lax.axis_index("c")
        p0 = wid * _PPW

        idx_copies = [
            pltpu.async_copy(
                src_hbm.at[b, pl.ds(p0, _PPW)],
                idx_v.at[b // 2, pl.ds((b % 2) * _PPW, _PPW)],
                isem,
            )
            for b in range(_B)
        ]
        pe_copy = pltpu.async_copy(pe_hbm.at[pl.ds(p0, _PPW)], pe_v, csem)
        for c in idx_copies:
            c.wait()
        gathers = [
            pltpu.async_copy(table_hbm.at[idx_v.at[j]], gbufs[j], gsems[j])
            for j in range(_B // 2)
        ]
        pe_copy.wait()

        stores = []
        for j in range(_B // 2):
            gathers[j].wait()
            gb = gbufs[j]

            @pl.loop(0, _PPW)
            def _(r, gb=gb):
                @pl.loop(0, _D, step=_LANES)
                def _(c, r=r, gb=gb):
                    p = pe_v[r, pl.ds(c, _LANES)]
                    sl0 = (r, pl.ds(c, _LANES))
                    sl1 = (r + _PPW, pl.ds(c, _LANES))
                    gb[sl0] = gb[sl0] * _SCALE + p
                    gb[sl1] = gb[sl1] * _SCALE + p

            for half in range(2):
                stores.append(
                    pltpu.async_copy(
                        gb.at[pl.ds(half * _PPW, _PPW)],
                        out_hbm.at[2 * j + half, pl.ds(p0, _PPW)],
                        osem,
                    )
                )
        for st in stores:
            st.wait()

    return k(table, src, pe2)


def kernel(src, table, pe):
    pe2 = pe.reshape(pe.shape[1], _D)[:_L]
    return _embed_sc(src, table, pe2)


# speedup vs baseline: 1.0702x; 1.0003x over previous
import functools
import math

import jax
import jax.numpy as jnp
from jax import lax
from jax.experimental import pallas as pl
from jax.experimental.pallas import tpu as pltpu
from jax.experimental.pallas import tpu_sc as plsc

_D = 128
_L = 2048
_B = 4
_NC = 2
_NS = 16
_NW = _NC * _NS
_PPW = _L // _NW
_LANES = 16
_SCALE = math.sqrt(float(_D))

_mesh = plsc.VectorSubcoreMesh(core_axis_name="c", subcore_axis_name="s")


@jax.jit
def _embed_sc(src, table, pe2):
    @functools.partial(
        pl.kernel,
        out_type=jax.ShapeDtypeStruct((_B, _L, _D), jnp.float32),
        mesh=_mesh,
        scratch_types=[
            pltpu.VMEM((_B // 2, 2 * _PPW), jnp.int32),
            [pltpu.VMEM((2 * _PPW, _D), jnp.float32) for _ in range(_B // 2)],
            pltpu.VMEM((_PPW, _D), jnp.float32),
            [pltpu.SemaphoreType.DMA for _ in range(_B // 2)],
            pltpu.SemaphoreType.DMA,
            pltpu.SemaphoreType.DMA,
            [pltpu.SemaphoreType.DMA for _ in range(_B // 2)],
        ],
    )
    def k(table_hbm, src_hbm, pe_hbm, out_hbm,
          idx_v, gbufs, pe_v, gsems, csem, osem, isems):
        wid = lax.axis_index("s") * _NC + lax.axis_index("c")
        p0 = wid * _PPW

        idx_copies = [
            pltpu.async_copy(
                src_hbm.at[b, pl.ds(p0, _PPW)],
                idx_v.at[b // 2, pl.ds((b % 2) * _PPW, _PPW)],
                isems[b // 2],
            )
            for b in range(_B)
        ]
        pe_copy = pltpu.async_copy(pe_hbm.at[pl.ds(p0, _PPW)], pe_v, csem)
        gathers = []
        for j in range(_B // 2):
            idx_copies[2 * j].wait()
            idx_copies[2 * j + 1].wait()
            gathers.append(
                pltpu.async_copy(table_hbm.at[idx_v.at[j]], gbufs[j], gsems[j])
            )
        pe_copy.wait()

        stores = []
        for j in range(_B // 2):
            gathers[j].wait()
            gb = gbufs[j]

            @pl.loop(0, _PPW)
            def _(r, gb=gb):
                @pl.loop(0, _D, step=_LANES)
                def _(c, r=r, gb=gb):
                    p = pe_v[r, pl.ds(c, _LANES)]
                    sl0 = (r, pl.ds(c, _LANES))
                    sl1 = (r + _PPW, pl.ds(c, _LANES))
                    gb[sl0] = gb[sl0] * _SCALE + p
                    gb[sl1] = gb[sl1] * _SCALE + p

            for half in range(2):
                stores.append(
                    pltpu.async_copy(
                        gb.at[pl.ds(half * _PPW, _PPW)],
                        out_hbm.at[2 * j + half, pl.ds(p0, _PPW)],
                        osem,
                    )
                )
        for st in stores:
            st.wait()

    return k(table, src, pe2)


def kernel(src, table, pe):
    pe2 = pe.reshape(pe.shape[1], _D)[:_L]
    return _embed_sc(src, table, pe2)


# speedup vs baseline: 1.0799x; 1.0091x over previous
import functools
import math

import jax
import jax.numpy as jnp
from jax import lax
from jax.experimental import pallas as pl
from jax.experimental.pallas import tpu as pltpu
from jax.experimental.pallas import tpu_sc as plsc

_D = 128
_L = 2048
_B = 4
_NC = 2
_NS = 16
_NW = _NC * _NS
_PPW = _L // _NW
_LANES = 16
_SCALE = math.sqrt(float(_D))

_mesh = plsc.VectorSubcoreMesh(core_axis_name="c", subcore_axis_name="s")


@jax.jit
def _embed_sc(src, table, pe2):
    @functools.partial(
        pl.kernel,
        out_type=jax.ShapeDtypeStruct((_B, _L, _D), jnp.float32),
        mesh=_mesh,
        scratch_types=[
            pltpu.VMEM((_B // 2, 2 * _PPW), jnp.int32),
            [pltpu.VMEM((2 * _PPW, _D), jnp.float32) for _ in range(_B // 2)],
            pltpu.VMEM((_PPW, _D), jnp.float32),
            [pltpu.SemaphoreType.DMA for _ in range(_B // 2)],
            pltpu.SemaphoreType.DMA,
            pltpu.SemaphoreType.DMA,
            [pltpu.SemaphoreType.DMA for _ in range(_B // 2)],
        ],
    )
    def k(table_hbm, src_hbm, pe_hbm, out_hbm,
          idx_v, gbufs, pe_v, gsems, csem, osem, isems):
        wid = lax.axis_index("s") * _NC + lax.axis_index("c")
        p0 = wid * _PPW

        idx_copies = [
            pltpu.async_copy(
                src_hbm.at[b, pl.ds(p0, _PPW)],
                idx_v.at[b // 2, pl.ds((b % 2) * _PPW, _PPW)],
                isems[b // 2],
            )
            for b in range(_B)
        ]
        pe_copy = pltpu.async_copy(pe_hbm.at[pl.ds(p0, _PPW)], pe_v, csem)
        gathers = []
        for j in range(_B // 2):
            idx_copies[2 * j].wait()
            idx_copies[2 * j + 1].wait()
            gathers.append(
                pltpu.async_copy(table_hbm.at[idx_v.at[j]], gbufs[j], gsems[j])
            )
        pe_copy.wait()

        stores = []
        sub = _PPW // 2
        for j in range(_B // 2):
            gathers[j].wait()
            gb = gbufs[j]

            for q in range(2):
                @pl.loop(q * sub, (q + 1) * sub)
                def _(r, gb=gb):
                    @pl.loop(0, _D, step=_LANES)
                    def _(c, r=r, gb=gb):
                        p = pe_v[r, pl.ds(c, _LANES)]
                        sl0 = (r, pl.ds(c, _LANES))
                        sl1 = (r + _PPW, pl.ds(c, _LANES))
                        gb[sl0] = gb[sl0] * _SCALE + p
                        gb[sl1] = gb[sl1] * _SCALE + p

                for half in range(2):
                    stores.append(
                        pltpu.async_copy(
                            gb.at[pl.ds(half * _PPW + q * sub, sub)],
                            out_hbm.at[2 * j + half, pl.ds(p0 + q * sub, sub)],
                            osem,
                        )
                    )
        for st in stores:
            st.wait()

    return k(table, src, pe2)


def kernel(src, table, pe):
    pe2 = pe.reshape(pe.shape[1], _D)[:_L]
    return _embed_sc(src, table, pe2)
